# baseline (speedup 1.0000x reference)
import jax
import jax.numpy as jnp
from jax import lax
from jax.experimental import pallas as pl
from jax.experimental.pallas import tpu as pltpu
from jax.experimental.pallas import tpu_sc as plsc

BATCH = 16384
VOCAB = 1000
EMBED = 16
BLOCK_N = 2048

NW = 32
SC_COLS = 256
CHUNK_R = 200
NCHUNK = VOCAB // CHUNK_R


def _body(w_ref, x_ref, o_ref):
    wb = w_ref[...].astype(jnp.bfloat16)
    xb = x_ref[...].astype(jnp.bfloat16)
    o_ref[...] = jax.lax.dot_general(
        wb, xb,
        dimension_numbers=(((1,), (0,)), ((), ())),
        preferred_element_type=jnp.float32,
        precision=jax.lax.Precision.DEFAULT,
    )


def _sc_body(x_hbm, o_hbm, buf, row):
    wid = lax.axis_index("s") * 2 + lax.axis_index("c")
    c0 = wid * SC_COLS

    def step(i, _):
        pltpu.sync_copy(x_hbm.at[pl.ds(i * CHUNK_R, CHUNK_R),
                                 pl.ds(c0, SC_COLS)], buf)
        return 0

    lax.fori_loop(0, NCHUNK, step, 0)
    row[...] = buf[0, 0:16]
    pltpu.sync_copy(row, o_hbm.at[wid])


def kernel(one_hot, weight):
    x_t = one_hot.T
    w_t = weight.T
    grid = (BATCH // BLOCK_N,)
    out_t = pl.pallas_call(
        _body,
        grid=grid,
        in_specs=[
            pl.BlockSpec((EMBED, VOCAB), lambda i: (0, 0)),
            pl.BlockSpec((VOCAB, BLOCK_N), lambda i: (0, i)),
        ],
        out_specs=pl.BlockSpec((EMBED, BLOCK_N), lambda i: (0, i)),
        out_shape=jax.ShapeDtypeStruct((EMBED, BATCH), jnp.float32),
        compiler_params=pltpu.CompilerParams(
            dimension_semantics=("arbitrary",),
        ),
    )(w_t, x_t)

    sc_probe = pl.kernel(
        _sc_body,
        out_type=jax.ShapeDtypeStruct((NW, 16), jnp.float32),
        mesh=plsc.VectorSubcoreMesh(core_axis_name="c", subcore_axis_name="s"),
        scratch_types=[
            pltpu.VMEM((CHUNK_R, SC_COLS), jnp.float32),
            pltpu.VMEM((16,), jnp.float32),
        ],
    )(x_t)

    out = out_t.T
    return out.at[0:NW, :].add(sc_probe * 1e-30)


# speedup vs baseline: 1.2144x; 1.2144x over previous
import jax
import jax.numpy as jnp
from jax import lax
from jax.experimental import pallas as pl
from jax.experimental.pallas import tpu as pltpu
from jax.experimental.pallas import tpu_sc as plsc

BATCH = 16384
VOCAB = 1000
EMBED = 16
BLOCK_N = 2048

SC_BATCH = 2048
TC_BATCH = BATCH - SC_BATCH
NSUB = 16
SC_COLS = SC_BATCH // NSUB
CHUNK_R = 200
NCHUNK = VOCAB // CHUNK_R


def _tc_body(w_ref, x_ref, o_ref):
    wb = w_ref[...].astype(jnp.bfloat16)
    xb = x_ref[...].astype(jnp.bfloat16)
    o_ref[...] = jax.lax.dot_general(
        wb, xb,
        dimension_numbers=(((1,), (0,)), ((), ())),
        preferred_element_type=jnp.float32,
        precision=jax.lax.Precision.DEFAULT,
    )


def _sc_body(x_hbm, w_hbm, o_hbm, buf, idx_v, rows_v, sems, gsem):
    wid = lax.axis_index("s")
    c0 = TC_BATCH + wid * SC_COLS

    def start(rc, b):
        pltpu.async_copy(
            x_hbm.at[pl.ds(rc * CHUNK_R, CHUNK_R), pl.ds(c0, SC_COLS)],
            buf.at[b],
            sems.at[b],
        )

    start(0, 0)
    accs = [jnp.zeros((16,), jnp.float32) for _ in range(8)]
    for rc in range(NCHUNK):
        b = rc % 2
        if rc + 1 < NCHUNK:
            start(rc + 1, (rc + 1) % 2)
        pltpu.make_async_copy(
            x_hbm.at[pl.ds(rc * CHUNK_R, CHUNK_R), pl.ds(c0, SC_COLS)],
            buf.at[b],
            sems.at[b],
        ).wait()

        def dec(v, acc):
            vf = (rc * CHUNK_R + v).astype(jnp.float32)
            vb = jnp.full((16,), vf, jnp.float32)
            return tuple(
                acc[j] + buf[b, v, pl.ds(16 * j, 16)] * vb for j in range(8)
            )

        accs = lax.fori_loop(0, CHUNK_R, dec, tuple(accs))
        accs = list(accs)

    for j in range(8):
        idx_v[pl.ds(16 * j, 16)] = accs[j].astype(jnp.int32)

    pltpu.async_copy(w_hbm.at[idx_v], rows_v, gsem).wait()
    pltpu.sync_copy(rows_v, o_hbm.at[pl.ds(wid * SC_COLS, SC_COLS), :])


WPAD = 128


def kernel(one_hot, weight):
    x_t = one_hot.T
    w_t = weight.T
    grid = (TC_BATCH // BLOCK_N,)
    out_t = pl.pallas_call(
        _tc_body,
        grid=grid,
        in_specs=[
            pl.BlockSpec((EMBED, VOCAB), lambda i: (0, 0)),
            pl.BlockSpec((VOCAB, BLOCK_N), lambda i: (0, i)),
        ],
        out_specs=pl.BlockSpec((EMBED, BLOCK_N), lambda i: (0, i)),
        out_shape=jax.ShapeDtypeStruct((EMBED, TC_BATCH), jnp.float32),
        compiler_params=pltpu.CompilerParams(
            dimension_semantics=("arbitrary",),
        ),
    )(w_t, x_t)

    sc_out = pl.kernel(
        _sc_body,
        out_type=jax.ShapeDtypeStruct((SC_BATCH, WPAD), jnp.float32),
        mesh=plsc.VectorSubcoreMesh(
            core_axis_name="c", subcore_axis_name="s", num_cores=1
        ),
        scratch_types=[
            pltpu.VMEM((2, CHUNK_R, SC_COLS), jnp.float32),
            pltpu.VMEM((SC_COLS,), jnp.int32),
            pltpu.VMEM((SC_COLS, WPAD), jnp.float32),
            pltpu.SemaphoreType.DMA((2,)),
            pltpu.SemaphoreType.DMA,
        ],
    )(x_t, jnp.pad(weight, ((0, 0), (0, WPAD - EMBED))))

    return jnp.concatenate([out_t.T, sc_out[:, :EMBED]], axis=0)


# speedup vs baseline: 1.2171x; 1.0023x over previous
import jax
import jax.numpy as jnp
from jax import lax
from jax.experimental import pallas as pl
from jax.experimental.pallas import tpu as pltpu
from jax.experimental.pallas import tpu_sc as plsc

BATCH = 16384
VOCAB = 1000
EMBED = 16
BLOCK_N = 2048

SC_BATCH = 2048
TC_BATCH = BATCH - SC_BATCH
NSUB = 16
SC_COLS = SC_BATCH // NSUB
CHUNK_R = 200
NCHUNK = VOCAB // CHUNK_R


def _tc_body(w_ref, x_ref, o_ref):
    wb = w_ref[...].astype(jnp.bfloat16)
    xb = x_ref[...].astype(jnp.bfloat16)
    o_ref[...] = jax.lax.dot_general(
        wb, xb,
        dimension_numbers=(((1,), (0,)), ((), ())),
        preferred_element_type=jnp.float32,
        precision=jax.lax.Precision.DEFAULT,
    )


def _sc_body(x_hbm, w_hbm, o_hbm, buf, idx_v, rows_v, sems, gsem):
    wid = lax.axis_index("s")
    c0 = TC_BATCH + wid * SC_COLS

    def start(rc, b):
        pltpu.async_copy(
            x_hbm.at[pl.ds(rc * CHUNK_R, CHUNK_R), pl.ds(c0, SC_COLS)],
            buf.at[b],
            sems.at[b],
        )

    start(0, 0)
    accs = [jnp.zeros((16,), jnp.float32) for _ in range(8)]
    for rc in range(NCHUNK):
        b = rc % 2
        if rc + 1 < NCHUNK:
            start(rc + 1, (rc + 1) % 2)
        pltpu.make_async_copy(
            x_hbm.at[pl.ds(rc * CHUNK_R, CHUNK_R), pl.ds(c0, SC_COLS)],
            buf.at[b],
            sems.at[b],
        ).wait()

        def dec(v, acc):
            vf = (rc * CHUNK_R + v).astype(jnp.float32)
            vb = jnp.full((16,), vf, jnp.float32)
            return tuple(
                acc[j] + buf[b, v, pl.ds(16 * j, 16)] * vb for j in range(8)
            )

        accs = lax.fori_loop(0, CHUNK_R, dec, tuple(accs))
        accs = list(accs)

    for j in range(8):
        idx_v[pl.ds(16 * j, 16)] = accs[j].astype(jnp.int32)

    pltpu.async_copy(w_hbm.at[idx_v], rows_v, gsem).wait()
    pltpu.sync_copy(rows_v, o_hbm.at[pl.ds(wid * SC_COLS, SC_COLS), :])


WPAD = 128


def kernel(one_hot, weight):
    x_t = one_hot.T
    w_t = weight.T
    grid = (TC_BATCH // BLOCK_N,)
    sc_out = pl.kernel(
        _sc_body,
        out_type=jax.ShapeDtypeStruct((SC_BATCH, WPAD), jnp.float32),
        mesh=plsc.VectorSubcoreMesh(
            core_axis_name="c", subcore_axis_name="s", num_cores=1
        ),
        scratch_types=[
            pltpu.VMEM((2, CHUNK_R, SC_COLS), jnp.float32),
            pltpu.VMEM((SC_COLS,), jnp.int32),
            pltpu.VMEM((SC_COLS, WPAD), jnp.float32),
            pltpu.SemaphoreType.DMA((2,)),
            pltpu.SemaphoreType.DMA,
        ],
    )(x_t, jnp.pad(weight, ((0, 0), (0, WPAD - EMBED))))

    out_t = pl.pallas_call(
        _tc_body,
        grid=grid,
        in_specs=[
            pl.BlockSpec((EMBED, VOCAB), lambda i: (0, 0)),
            pl.BlockSpec((VOCAB, BLOCK_N), lambda i: (0, i)),
        ],
        out_specs=pl.BlockSpec((EMBED, BLOCK_N), lambda i: (0, i)),
        out_shape=jax.ShapeDtypeStruct((EMBED, TC_BATCH), jnp.float32),
        compiler_params=pltpu.CompilerParams(
            dimension_semantics=("arbitrary",),
        ),
    )(w_t, x_t)

    return jnp.concatenate([out_t.T, sc_out[:, :EMBED]], axis=0)


# speedup vs baseline: 2.3008x; 1.8903x over previous
import jax
import jax.numpy as jnp
from jax.experimental import pallas as pl
from jax.experimental.pallas import tpu as pltpu

BATCH = 16384
VOCAB = 1000
EMBED = 16
CHUNK_N = 1024
NCHUNK = BATCH // CHUNK_N
NBUF = 4


def _body(w_ref, x_hbm, o_ref, xbuf, sems):
    wb = w_ref[...].astype(jnp.bfloat16)

    def start(c, b):
        pltpu.make_async_copy(
            x_hbm.at[:, pl.ds(c * CHUNK_N, CHUNK_N)],
            xbuf.at[b],
            sems.at[b],
        ).start()

    for c in range(NBUF):
        start(c, c)

    for c in range(NCHUNK):
        b = c % NBUF
        pltpu.make_async_copy(
            x_hbm.at[:, pl.ds(c * CHUNK_N, CHUNK_N)],
            xbuf.at[b],
            sems.at[b],
        ).wait()
        xb = xbuf[b].astype(jnp.bfloat16)
        o_ref[:, pl.ds(c * CHUNK_N, CHUNK_N)] = jax.lax.dot_general(
            wb, xb,
            dimension_numbers=(((1,), (0,)), ((), ())),
            preferred_element_type=jnp.float32,
            precision=jax.lax.Precision.DEFAULT,
        )
        if c + NBUF < NCHUNK:
            start(c + NBUF, b)


def kernel(one_hot, weight):
    x_t = one_hot.T
    w_t = weight.T
    out_t = pl.pallas_call(
        _body,
        in_specs=[
            pl.BlockSpec(memory_space=pltpu.MemorySpace.VMEM),
            pl.BlockSpec(memory_space=pltpu.MemorySpace.HBM),
        ],
        out_specs=pl.BlockSpec(memory_space=pltpu.MemorySpace.VMEM),
        out_shape=jax.ShapeDtypeStruct((EMBED, BATCH), jnp.float32),
        scratch_shapes=[
            pltpu.VMEM((NBUF, VOCAB, CHUNK_N), jnp.float32),
            pltpu.SemaphoreType.DMA((NBUF,)),
        ],
    )(w_t, x_t)
    return out_t.T


# speedup vs baseline: 2.3597x; 1.0256x over previous
import jax
import jax.numpy as jnp
from jax.experimental import pallas as pl
from jax.experimental.pallas import tpu as pltpu

BATCH = 16384
VOCAB = 1000
EMBED = 16
BLOCK_N = 2048


def _body(w_ref, x_ref, o_ref):
    wb = w_ref[...].astype(jnp.bfloat16)
    xb = x_ref[...].astype(jnp.bfloat16)
    o_ref[...] = jax.lax.dot_general(
        wb, xb,
        dimension_numbers=(((1,), (0,)), ((), ())),
        preferred_element_type=jnp.float32,
        precision=jax.lax.Precision.DEFAULT,
    )


def kernel(one_hot, weight):
    x_t = one_hot.T
    w_t = weight.T
    grid = (BATCH // BLOCK_N,)
    out_t = pl.pallas_call(
        _body,
        grid=grid,
        in_specs=[
            pl.BlockSpec((EMBED, VOCAB), lambda i: (0, 0)),
            pl.BlockSpec((VOCAB, BLOCK_N), lambda i: (0, i)),
        ],
        out_specs=pl.BlockSpec((EMBED, BLOCK_N), lambda i: (0, i)),
        out_shape=jax.ShapeDtypeStruct((EMBED, BATCH), jnp.float32),
        compiler_params=pltpu.CompilerParams(
            dimension_semantics=("arbitrary",),
        ),
    )(w_t, x_t)
    return out_t.T
